# baseline (speedup 1.0000x reference)
import jax
import jax.numpy as jnp
from jax.experimental import pallas as pl
from jax.experimental.pallas import tpu as pltpu

_R = 256


def _aug_rows(t):
    n = t[0:1, :] * t[0:1, :] + t[1:2, :] * t[1:2, :] + t[2:3, :] * t[2:3, :]
    ones = jnp.ones_like(n)
    r = jnp.concatenate([t, ones, n], axis=0)
    rhi = r.astype(jnp.bfloat16)
    rlo = (r - rhi.astype(jnp.float32)).astype(jnp.bfloat16)
    pad = jnp.zeros_like(n, dtype=jnp.bfloat16)
    return jnp.concatenate([rhi, rlo, rhi, pad], axis=0)


def _aug_cols(xb):
    nb = (xb[:, 0:1] * xb[:, 0:1] + xb[:, 1:2] * xb[:, 1:2]
          + xb[:, 2:3] * xb[:, 2:3])
    ones = jnp.ones_like(nb)
    l = jnp.concatenate([xb * (-2.0), nb, ones], axis=1)
    lhi = l.astype(jnp.bfloat16)
    llo = (l - lhi.astype(jnp.float32)).astype(jnp.bfloat16)
    pad = jnp.zeros_like(nb, dtype=jnp.bfloat16)
    return jnp.concatenate([lhi, lhi, llo, pad], axis=1)


def _chamfer_tc_kernel(x1_ref, x2_ref, x1t_ref, x2t_ref, d1_ref, d2_ref,
                       r1_ref, r2_ref):
    i = pl.program_id(1)

    @pl.when(i == 0)
    def _():
        r1_ref[...] = _aug_rows(x1t_ref[0])
        r2_ref[...] = _aug_rows(x2t_ref[0])

    dims = (((1,), (0,)), ((), ()))
    da = jax.lax.dot_general(_aug_cols(x1_ref[0]), r2_ref[...],
                             dimension_numbers=dims,
                             preferred_element_type=jnp.float32)
    db = jax.lax.dot_general(_aug_cols(x2_ref[0]), r1_ref[...],
                             dimension_numbers=dims,
                             preferred_element_type=jnp.float32)
    cma = jnp.min(da, axis=0)
    cmb = jnp.min(db, axis=0)

    @pl.when(i == 0)
    def _():
        d2_ref[0, 0, :] = cma
        d1_ref[0, 0, :] = cmb

    @pl.when(i != 0)
    def _():
        d2_ref[0, 0, :] = jnp.minimum(d2_ref[0, 0, :], cma)
        d1_ref[0, 0, :] = jnp.minimum(d1_ref[0, 0, :], cmb)


def kernel(xyz1, xyz2):
    B, N, _ = xyz1.shape
    M = xyz2.shape[1]
    x1t = jnp.swapaxes(xyz1, 1, 2)
    x2t = jnp.swapaxes(xyz2, 1, 2)
    d1, d2 = pl.pallas_call(
        _chamfer_tc_kernel,
        grid=(B, N // _R),
        in_specs=[
            pl.BlockSpec((1, _R, 3), lambda b, i: (b, i, 0)),
            pl.BlockSpec((1, _R, 3), lambda b, i: (b, i, 0)),
            pl.BlockSpec((1, 3, N), lambda b, i: (b, 0, 0)),
            pl.BlockSpec((1, 3, M), lambda b, i: (b, 0, 0)),
        ],
        out_specs=[
            pl.BlockSpec((1, 1, N), lambda b, i: (b, 0, 0)),
            pl.BlockSpec((1, 1, M), lambda b, i: (b, 0, 0)),
        ],
        out_shape=[
            jax.ShapeDtypeStruct((B, 1, N), jnp.float32),
            jax.ShapeDtypeStruct((B, 1, M), jnp.float32),
        ],
        scratch_shapes=[
            pltpu.VMEM((16, N), jnp.bfloat16),
            pltpu.VMEM((16, M), jnp.bfloat16),
        ],
        compiler_params=pltpu.CompilerParams(
            dimension_semantics=("parallel", "arbitrary")),
    )(xyz1, xyz2, x1t, x2t)
    return d1.reshape(B, N), d2.reshape(B, M)


# speedup vs baseline: 1.3971x; 1.3971x over previous
import jax
import jax.numpy as jnp
from jax.experimental import pallas as pl
from jax.experimental.pallas import tpu as pltpu

_R = 512


def _chamfer_tc_kernel(x1_ref, x2t_ref, d1_ref, d2_ref):
    ib = pl.program_id(1)
    x1 = x1_ref[0]
    x2 = x2t_ref[0]
    n1 = (x1[:, 0:1] * x1[:, 0:1] + x1[:, 1:2] * x1[:, 1:2]
          + x1[:, 2:3] * x1[:, 2:3])
    n2 = (x2[0:1, :] * x2[0:1, :] + x2[1:2, :] * x2[1:2, :]
          + x2[2:3, :] * x2[2:3, :])
    ones_r = jnp.ones_like(n1)
    lhs = jnp.concatenate([x1 * (-2.0), n1, ones_r], axis=1)
    ones_m = jnp.ones_like(n2)
    rhs = jnp.concatenate([x2, ones_m, n2], axis=0)
    lhs_hi = lhs.astype(jnp.bfloat16)
    lhs_lo = (lhs - lhs_hi.astype(jnp.float32)).astype(jnp.bfloat16)
    rhs_hi = rhs.astype(jnp.bfloat16)
    rhs_lo = (rhs - rhs_hi.astype(jnp.float32)).astype(jnp.bfloat16)
    lhs_aug = jnp.concatenate([lhs_hi, lhs_hi, lhs_lo], axis=1)
    rhs_aug = jnp.concatenate([rhs_hi, rhs_lo, rhs_hi], axis=0)
    d = jax.lax.dot_general(
        lhs_aug, rhs_aug,
        dimension_numbers=(((1,), (0,)), ((), ())),
        preferred_element_type=jnp.float32,
    )
    d1_ref[0, 0, pl.ds(ib * _R, _R)] = jnp.min(d, axis=1)
    colmin = jnp.min(d, axis=0)

    @pl.when(ib == 0)
    def _():
        d2_ref[0, 0, :] = colmin

    @pl.when(ib != 0)
    def _():
        d2_ref[0, 0, :] = jnp.minimum(d2_ref[0, 0, :], colmin)


def kernel(xyz1, xyz2):
    B, N, _ = xyz1.shape
    M = xyz2.shape[1]
    x2t = jnp.swapaxes(xyz2, 1, 2)
    d1, d2 = pl.pallas_call(
        _chamfer_tc_kernel,
        grid=(B, N // _R),
        in_specs=[
            pl.BlockSpec((1, _R, 3), lambda b, i: (b, i, 0)),
            pl.BlockSpec((1, 3, M), lambda b, i: (b, 0, 0)),
        ],
        out_specs=[
            pl.BlockSpec((1, 1, N), lambda b, i: (b, 0, 0)),
            pl.BlockSpec((1, 1, M), lambda b, i: (b, 0, 0)),
        ],
        out_shape=[
            jax.ShapeDtypeStruct((B, 1, N), jnp.float32),
            jax.ShapeDtypeStruct((B, 1, M), jnp.float32),
        ],
        compiler_params=pltpu.CompilerParams(
            dimension_semantics=("parallel", "arbitrary")),
    )(xyz1, x2t)
    return d1.reshape(B, N), d2.reshape(B, M)


# speedup vs baseline: 1.4194x; 1.0160x over previous
import jax
import jax.numpy as jnp
from jax.experimental import pallas as pl
from jax.experimental.pallas import tpu as pltpu

_R = 1024


def _chamfer_tc_kernel(x1_ref, x2t_ref, d1_ref, d2_ref):
    ib = pl.program_id(1)
    x1 = x1_ref[0]
    x2 = x2t_ref[0]
    n1 = (x1[:, 0:1] * x1[:, 0:1] + x1[:, 1:2] * x1[:, 1:2]
          + x1[:, 2:3] * x1[:, 2:3])
    n2 = (x2[0:1, :] * x2[0:1, :] + x2[1:2, :] * x2[1:2, :]
          + x2[2:3, :] * x2[2:3, :])
    ones_r = jnp.ones_like(n1)
    lhs = jnp.concatenate([x1 * (-2.0), n1, ones_r], axis=1)
    ones_m = jnp.ones_like(n2)
    rhs = jnp.concatenate([x2, ones_m, n2], axis=0)
    lhs_hi = lhs.astype(jnp.bfloat16)
    lhs_lo = (lhs - lhs_hi.astype(jnp.float32)).astype(jnp.bfloat16)
    rhs_hi = rhs.astype(jnp.bfloat16)
    rhs_lo = (rhs - rhs_hi.astype(jnp.float32)).astype(jnp.bfloat16)
    lhs_aug = jnp.concatenate([lhs_hi, lhs_hi, lhs_lo], axis=1)
    rhs_aug = jnp.concatenate([rhs_hi, rhs_lo, rhs_hi], axis=0)
    d = jax.lax.dot_general(
        lhs_aug, rhs_aug,
        dimension_numbers=(((1,), (0,)), ((), ())),
        preferred_element_type=jnp.float32,
    )
    d1_ref[0, 0, pl.ds(ib * _R, _R)] = jnp.min(d, axis=1)
    colmin = jnp.min(d, axis=0)

    @pl.when(ib == 0)
    def _():
        d2_ref[0, 0, :] = colmin

    @pl.when(ib != 0)
    def _():
        d2_ref[0, 0, :] = jnp.minimum(d2_ref[0, 0, :], colmin)


def kernel(xyz1, xyz2):
    B, N, _ = xyz1.shape
    M = xyz2.shape[1]
    x2t = jnp.swapaxes(xyz2, 1, 2)
    d1, d2 = pl.pallas_call(
        _chamfer_tc_kernel,
        grid=(B, N // _R),
        in_specs=[
            pl.BlockSpec((1, _R, 3), lambda b, i: (b, i, 0)),
            pl.BlockSpec((1, 3, M), lambda b, i: (b, 0, 0)),
        ],
        out_specs=[
            pl.BlockSpec((1, 1, N), lambda b, i: (b, 0, 0)),
            pl.BlockSpec((1, 1, M), lambda b, i: (b, 0, 0)),
        ],
        out_shape=[
            jax.ShapeDtypeStruct((B, 1, N), jnp.float32),
            jax.ShapeDtypeStruct((B, 1, M), jnp.float32),
        ],
        compiler_params=pltpu.CompilerParams(
            dimension_semantics=("parallel", "arbitrary")),
    )(xyz1, x2t)
    return d1.reshape(B, N), d2.reshape(B, M)


# speedup vs baseline: 1.4281x; 1.0061x over previous
import jax
import jax.numpy as jnp
from jax.experimental import pallas as pl
from jax.experimental.pallas import tpu as pltpu

_R = 2048


def _chamfer_tc_kernel(x1_ref, x2t_ref, d1_ref, d2_ref):
    ib = pl.program_id(1)
    x1 = x1_ref[0]
    x2 = x2t_ref[0]
    n1 = (x1[:, 0:1] * x1[:, 0:1] + x1[:, 1:2] * x1[:, 1:2]
          + x1[:, 2:3] * x1[:, 2:3])
    n2 = (x2[0:1, :] * x2[0:1, :] + x2[1:2, :] * x2[1:2, :]
          + x2[2:3, :] * x2[2:3, :])
    ones_r = jnp.ones_like(n1)
    lhs = jnp.concatenate([x1 * (-2.0), n1, ones_r], axis=1)
    ones_m = jnp.ones_like(n2)
    rhs = jnp.concatenate([x2, ones_m, n2], axis=0)
    lhs_hi = lhs.astype(jnp.bfloat16)
    lhs_lo = (lhs - lhs_hi.astype(jnp.float32)).astype(jnp.bfloat16)
    rhs_hi = rhs.astype(jnp.bfloat16)
    rhs_lo = (rhs - rhs_hi.astype(jnp.float32)).astype(jnp.bfloat16)
    lhs_aug = jnp.concatenate([lhs_hi, lhs_hi, lhs_lo], axis=1)
    rhs_aug = jnp.concatenate([rhs_hi, rhs_lo, rhs_hi], axis=0)
    d = jax.lax.dot_general(
        lhs_aug, rhs_aug,
        dimension_numbers=(((1,), (0,)), ((), ())),
        preferred_element_type=jnp.float32,
    )
    d1_ref[0, 0, pl.ds(ib * _R, _R)] = jnp.min(d, axis=1)
    colmin = jnp.min(d, axis=0)

    @pl.when(ib == 0)
    def _():
        d2_ref[0, 0, :] = colmin

    @pl.when(ib != 0)
    def _():
        d2_ref[0, 0, :] = jnp.minimum(d2_ref[0, 0, :], colmin)


def kernel(xyz1, xyz2):
    B, N, _ = xyz1.shape
    M = xyz2.shape[1]
    x2t = jnp.swapaxes(xyz2, 1, 2)
    d1, d2 = pl.pallas_call(
        _chamfer_tc_kernel,
        grid=(B, N // _R),
        in_specs=[
            pl.BlockSpec((1, _R, 3), lambda b, i: (b, i, 0)),
            pl.BlockSpec((1, 3, M), lambda b, i: (b, 0, 0)),
        ],
        out_specs=[
            pl.BlockSpec((1, 1, N), lambda b, i: (b, 0, 0)),
            pl.BlockSpec((1, 1, M), lambda b, i: (b, 0, 0)),
        ],
        out_shape=[
            jax.ShapeDtypeStruct((B, 1, N), jnp.float32),
            jax.ShapeDtypeStruct((B, 1, M), jnp.float32),
        ],
        compiler_params=pltpu.CompilerParams(
            dimension_semantics=("parallel", "arbitrary")),
    )(xyz1, x2t)
    return d1.reshape(B, N), d2.reshape(B, M)
